# speedup vs baseline: 3.2289x; 3.2289x over previous
import functools

import jax
import jax.numpy as jnp
from jax import lax
from jax.experimental import pallas as pl
from jax.experimental.pallas import tpu as pltpu
from jax.experimental.pallas import tpu_sc as plsc

N_NODES = 10000
N_EDGES = 320000
D_FEAT = 128

NC = 2
NS = 16
NW = NC * NS

CH = 128
K = 8
SB = 10
CPW = K * SB
NCHT = NW * CPW
E_PAD = NCHT * CH

NP = 10240
ZROWS = NP // NS


def _sc_body(x_hbm, src_hbm, dst_hbm, zeros_hbm, p_hbm,
             src_b, dst_b, rows, acc, sem):
    c = lax.axis_index("c")
    s = lax.axis_index("s")
    wid = s * NC + c

    pltpu.sync_copy(zeros_hbm.at[pl.ds(s * ZROWS, ZROWS)],
                    acc.at[pl.ds(s * ZROWS, ZROWS)])
    plsc.subcore_barrier()

    wbase = wid * CPW

    @pl.loop(0, SB)
    def _(b):
        cb = wbase + b * K
        pltpu.sync_copy(src_hbm.at[pl.ds(cb, K)], src_b)
        pltpu.sync_copy(dst_hbm.at[pl.ds(cb, K)], dst_b)
        for j in range(K):
            pltpu.async_copy(x_hbm.at[src_b.at[j]], rows, sem).wait()
            pltpu.sync_copy(rows, acc.at[dst_b.at[j]], add=True)

    plsc.subcore_barrier()
    pltpu.sync_copy(acc.at[pl.ds(s * ZROWS, ZROWS)],
                    p_hbm.at[c, pl.ds(s * ZROWS, ZROWS)])


@jax.jit
def _sc_scatter(x, src2, dst2, zeros):
    mesh = plsc.VectorSubcoreMesh(core_axis_name="c", subcore_axis_name="s")
    run = pl.kernel(
        _sc_body,
        out_type=jax.ShapeDtypeStruct((NC, NP, D_FEAT), jnp.float32),
        mesh=mesh,
        scratch_types=[
            pltpu.VMEM((K, CH), jnp.int32),
            pltpu.VMEM((K, CH), jnp.int32),
            pltpu.VMEM((CH, D_FEAT), jnp.float32),
            pltpu.VMEM_SHARED((NP, D_FEAT), jnp.float32),
            pltpu.SemaphoreType.DMA,
        ],
    )
    return run(x, src2, dst2, zeros)


def _combine_body(p_ref, o_ref):
    o_ref[...] = p_ref[0, :N_NODES, :] + p_ref[1, :N_NODES, :]


@jax.jit
def _combine(p):
    return pl.pallas_call(
        _combine_body,
        out_shape=jax.ShapeDtypeStruct((N_NODES, D_FEAT), jnp.float32),
    )(p)


def kernel(x, edge_index):
    src = edge_index[0]
    dst = edge_index[1]
    pad = E_PAD - N_EDGES
    src_p = jnp.concatenate([src, jnp.zeros((pad,), jnp.int32)])
    dst_p = jnp.concatenate([dst, jnp.full((pad,), NP - 1, jnp.int32)])
    src2 = src_p.reshape(NCHT, CH)
    dst2 = dst_p.reshape(NCHT, CH)
    zeros = jnp.zeros((NP, D_FEAT), jnp.float32)
    p = _sc_scatter(x, src2, dst2, zeros)
    return _combine(p)


# speedup vs baseline: 3.5213x; 1.0905x over previous
import jax
import jax.numpy as jnp
from jax import lax
from jax.experimental import pallas as pl
from jax.experimental.pallas import tpu as pltpu
from jax.experimental.pallas import tpu_sc as plsc

N_NODES = 10000
N_EDGES = 320000
D_FEAT = 128

NC = 2
NS = 16
NW = NC * NS

CH = 128
K = 8
SB = 10
CPW = K * SB
NCHT = NW * CPW
E_PAD = NCHT * CH

NP = 10240
ZROWS = NP // NS
TRASH = NP - 1


def _sc_body(x_hbm, src_hbm, dst_hbm, p_hbm,
             src_i, dst_i, rows_a, rows_b, acc, sem_a, sem_b):
    c = lax.axis_index("c")
    s = lax.axis_index("s")
    wid = s * NC + c
    wbase = wid * CPW

    @pl.loop(0, CH)
    def _(r):
        for c0 in range(0, CH, 16):
            rows_a[r, pl.ds(c0, 16)] = jnp.zeros((16,), jnp.float32)

    @pl.loop(0, ZROWS // CH)
    def _(i):
        pltpu.sync_copy(rows_a, acc.at[pl.ds(s * ZROWS + i * CH, CH)])

    plsc.subcore_barrier()

    rows = [rows_a, rows_b]
    sems = [sem_a, sem_b]

    @pl.loop(0, SB)
    def _(b):
        cb = wbase + b * K
        pltpu.sync_copy(src_hbm.at[pl.ds(cb, K)], src_i)
        pltpu.sync_copy(dst_hbm.at[pl.ds(cb, K)], dst_i)
        pltpu.make_async_copy(x_hbm.at[src_i.at[0]], rows[0], sems[0]).start()
        for j in range(K):
            if j + 1 < K:
                pltpu.make_async_copy(x_hbm.at[src_i.at[j + 1]],
                                      rows[(j + 1) % 2], sems[(j + 1) % 2]).start()
            pltpu.make_async_copy(x_hbm.at[src_i.at[j]],
                                  rows[j % 2], sems[j % 2]).wait()
            pltpu.sync_copy(rows[j % 2], acc.at[dst_i.at[j]], add=True)

    plsc.subcore_barrier()
    pltpu.sync_copy(acc.at[pl.ds(s * ZROWS, ZROWS)],
                    p_hbm.at[c, pl.ds(s * ZROWS, ZROWS)])


@jax.jit
def _sc_scatter(x, src2, dst2):
    mesh = plsc.VectorSubcoreMesh(core_axis_name="c", subcore_axis_name="s")
    run = pl.kernel(
        _sc_body,
        out_type=jax.ShapeDtypeStruct((NC, NP, D_FEAT), jnp.float32),
        mesh=mesh,
        scratch_types=[
            pltpu.VMEM((K, CH), jnp.int32),
            pltpu.VMEM((K, CH), jnp.int32),
            pltpu.VMEM((CH, D_FEAT), jnp.float32),
            pltpu.VMEM((CH, D_FEAT), jnp.float32),
            pltpu.VMEM_SHARED((NP, D_FEAT), jnp.float32),
            pltpu.SemaphoreType.DMA,
            pltpu.SemaphoreType.DMA,
        ],
    )
    return run(x, src2, dst2)


def _combine_body(p_ref, o_ref):
    o_ref[...] = p_ref[0, :N_NODES, :] + p_ref[1, :N_NODES, :]


@jax.jit
def _combine(p):
    return pl.pallas_call(
        _combine_body,
        out_shape=jax.ShapeDtypeStruct((N_NODES, D_FEAT), jnp.float32),
    )(p)


def kernel(x, edge_index):
    src = edge_index[0]
    dst = edge_index[1]
    pad = E_PAD - N_EDGES
    src_p = jnp.concatenate([src, jnp.zeros((pad,), jnp.int32)])
    dst_p = jnp.concatenate([dst, jnp.full((pad,), TRASH, jnp.int32)])
    src2 = src_p.reshape(NCHT, CH)
    dst2 = dst_p.reshape(NCHT, CH)
    p = _sc_scatter(x, src2, dst2)
    return _combine(p)


# speedup vs baseline: 11.6464x; 3.3074x over previous
import jax
import jax.numpy as jnp
from jax import lax
from jax.experimental import pallas as pl
from jax.experimental.pallas import tpu as pltpu
from jax.experimental.pallas import tpu_sc as plsc

N_NODES = 10000
N_EDGES = 320000
D_FEAT = 128

NC = 2
NS = 16
NW = NC * NS

CH = 128
K = 8
SB = 10
CPW = K * SB
NCHT = NW * CPW
E_PAD = NCHT * CH

NP = 10240
ZROWS = NP // NS
TRASH = NP - 1


def _sc_body(x_hbm, src_hbm, dst_hbm, p_hbm,
             src_i, dst_i, rows_a, rows_b, acc, sem_a, sem_b):
    c = lax.axis_index("c")
    s = lax.axis_index("s")
    wid = s * NC + c
    wbase = wid * CPW

    @pl.loop(0, CH)
    def _(r):
        for c0 in range(0, CH, 16):
            rows_a[r, pl.ds(c0, 16)] = jnp.zeros((16,), jnp.float32)

    @pl.loop(0, ZROWS // CH)
    def _(i):
        pltpu.sync_copy(rows_a, acc.at[pl.ds(s * ZROWS + i * CH, CH)])

    plsc.subcore_barrier()

    rows = [rows_a, rows_b]
    sems = [sem_a, sem_b]

    @pl.loop(0, SB)
    def _(b):
        cb = wbase + b * K
        pltpu.sync_copy(src_hbm.at[pl.ds(cb, K)], src_i)
        pltpu.sync_copy(dst_hbm.at[pl.ds(cb, K)], dst_i)
        pltpu.make_async_copy(x_hbm.at[src_i.at[0]], rows[0], sems[0]).start()
        for j in range(K):
            if j + 1 < K:
                pltpu.make_async_copy(x_hbm.at[src_i.at[j + 1]],
                                      rows[(j + 1) % 2], sems[(j + 1) % 2]).start()
            pltpu.make_async_copy(x_hbm.at[src_i.at[j]],
                                  rows[j % 2], sems[j % 2]).wait()
            pltpu.sync_copy(rows[j % 2], acc.at[dst_i.at[j]], add=True)

    plsc.subcore_barrier()
    pltpu.sync_copy(acc.at[pl.ds(s * ZROWS, ZROWS)],
                    p_hbm.at[c, pl.ds(s * ZROWS, ZROWS)])


@jax.jit
def _sc_scatter(x, src2, dst2):
    mesh = plsc.VectorSubcoreMesh(core_axis_name="c", subcore_axis_name="s")
    run = pl.kernel(
        _sc_body,
        out_type=jax.ShapeDtypeStruct((NC, NP, D_FEAT), jnp.float32),
        mesh=mesh,
        scratch_types=[
            pltpu.VMEM((K, CH), jnp.int32),
            pltpu.VMEM((K, CH), jnp.int32),
            pltpu.VMEM((CH, D_FEAT), jnp.float32),
            pltpu.VMEM((CH, D_FEAT), jnp.float32),
            pltpu.VMEM_SHARED((NP, D_FEAT), jnp.float32),
            pltpu.SemaphoreType.DMA,
            pltpu.SemaphoreType.DMA,
        ],
    )
    return run(x, src2, dst2)


def _combine_body(p_ref, o_ref):
    o_ref[...] = p_ref[0, :N_NODES, :] + p_ref[1, :N_NODES, :]


@jax.jit
def _combine(p):
    return pl.pallas_call(
        _combine_body,
        out_shape=jax.ShapeDtypeStruct((N_NODES, D_FEAT), jnp.float32),
    )(p)


def kernel(x, edge_index):
    src = edge_index[0]
    dst = edge_index[1]
    pad = E_PAD - N_EDGES
    pad_dst = N_NODES + (jnp.arange(pad, dtype=jnp.int32) % (NP - N_NODES))
    pad_src = jnp.arange(pad, dtype=jnp.int32) % N_NODES
    src_p = jnp.concatenate([src, pad_src])
    dst_p = jnp.concatenate([dst, pad_dst])
    src2 = src_p.reshape(NCHT, CH)
    dst2 = dst_p.reshape(NCHT, CH)
    p = _sc_scatter(x, src2, dst2)
    return _combine(p)
